# Initial kernel scaffold; baseline (speedup 1.0000x reference)
import functools

import jax
import jax.numpy as jnp
from jax import lax
from jax.experimental import pallas as pl
from jax.experimental.pallas import tpu as pltpu
from jax.experimental.pallas import tpu_sc as plsc

_B = 16384
_STORE_VOCAB = 2
_LOC_VOCAB = 3
_NCOMBO = _STORE_VOCAB * _LOC_VOCAB

_info = plsc.get_sparse_core_info()
_NC, _NS, _L = _info.num_cores, _info.num_subcores, _info.num_lanes
_NW = _NC * _NS
_BPW = _B // _NW


def _table_body(se, le, w1, b1, w2, b2, out):
    xs = jnp.repeat(se[...], _LOC_VOCAB, axis=0)
    xl = jnp.tile(le[...], (_STORE_VOCAB, 1))
    x = jnp.concatenate([xs, xl], axis=1)
    h = jnp.maximum(
        jnp.dot(x, w1[...], preferred_element_type=jnp.float32) + b1[...], 0.0)
    o = jnp.maximum(
        jnp.dot(h, w2[...], preferred_element_type=jnp.float32) + b2[...], 0.0)
    out[...] = jnp.concatenate(
        [o, jnp.zeros((8 - _NCOMBO, 1), jnp.float32)], axis=0)


_table_call = pl.pallas_call(
    _table_body,
    out_shape=jax.ShapeDtypeStruct((8, 1), jnp.float32),
)


def _lookup_body(table_hbm, store_hbm, loc_hbm, out_hbm,
                 table_v, store_v, loc_v, out_v):
    wid = lax.axis_index("s") * _NC + lax.axis_index("c")
    base = wid * _BPW
    pltpu.sync_copy(table_hbm, table_v)
    pltpu.sync_copy(store_hbm.at[pl.ds(base, _BPW)], store_v)
    pltpu.sync_copy(loc_hbm.at[pl.ds(base, _BPW)], loc_v)
    for i in range(_BPW // _L):
        s = store_v[pl.ds(i * _L, _L)]
        l = loc_v[pl.ds(i * _L, _L)]
        idx = s * _LOC_VOCAB + l
        out_v[pl.ds(i * _L, _L)] = plsc.load_gather(table_v, [idx])
    pltpu.sync_copy(out_v, out_hbm.at[pl.ds(base, _BPW)])


_lookup_call = pl.kernel(
    _lookup_body,
    mesh=plsc.VectorSubcoreMesh(core_axis_name="c", subcore_axis_name="s"),
    out_type=jax.ShapeDtypeStruct((_B,), jnp.float32),
    scratch_types=[
        pltpu.VMEM((8,), jnp.float32),
        pltpu.VMEM((_BPW,), jnp.int32),
        pltpu.VMEM((_BPW,), jnp.int32),
        pltpu.VMEM((_BPW,), jnp.float32),
    ],
)


def kernel(store, loc, store_emb, loc_emb, W1, b1, W2, b2):
    table = _table_call(store_emb, loc_emb, W1,
                        b1.reshape(1, -1), W2, b2.reshape(1, -1))
    out = _lookup_call(table.reshape(8),
                       store.astype(jnp.int32), loc.astype(jnp.int32))
    return out.reshape(_B, 1)


# speedup vs baseline: 4.6630x; 4.6630x over previous
import functools

import jax
import jax.numpy as jnp
from jax import lax
from jax.experimental import pallas as pl
from jax.experimental.pallas import tpu as pltpu
from jax.experimental.pallas import tpu_sc as plsc

_B = 16384
_STORE_VOCAB = 2
_LOC_VOCAB = 3
_NCOMBO = _STORE_VOCAB * _LOC_VOCAB

_NC, _NS, _L = 2, 16, 16
_NW = _NC * _NS
_BPW = _B // _NW


def _table_body(se, le, w1, b1, w2, b2, out):
    xs = jnp.repeat(se[...], _LOC_VOCAB, axis=0)
    xl = jnp.tile(le[...], (_STORE_VOCAB, 1))
    x = jnp.concatenate([xs, xl], axis=1)
    h = jnp.maximum(
        jnp.dot(x, w1[...], preferred_element_type=jnp.float32) + b1[...], 0.0)
    o = jnp.maximum(
        jnp.dot(h, w2[...], preferred_element_type=jnp.float32) + b2[...], 0.0)
    o16 = jnp.broadcast_to(o, (_NCOMBO, _L))
    out[...] = jnp.concatenate(
        [o16, jnp.zeros((8 - _NCOMBO, _L), jnp.float32)], axis=0)


_table_call = pl.pallas_call(
    _table_body,
    out_shape=jax.ShapeDtypeStruct((8, _L), jnp.float32),
)


def _lookup_body(table_hbm, store_hbm, loc_hbm, out_hbm,
                 table_v, store_v, loc_v, out_v):
    wid = lax.axis_index("s") * _NC + lax.axis_index("c")
    base = wid * _BPW
    pltpu.sync_copy(table_hbm, table_v)
    pltpu.sync_copy(store_hbm.at[pl.ds(base, _BPW)], store_v)
    pltpu.sync_copy(loc_hbm.at[pl.ds(base, _BPW)], loc_v)
    t = [table_v[k] for k in range(_NCOMBO)]
    for i in range(_BPW // _L):
        s = store_v[pl.ds(i * _L, _L)]
        l = loc_v[pl.ds(i * _L, _L)]
        idx = s * _LOC_VOCAB + l
        vals = t[_NCOMBO - 1]
        for k in range(_NCOMBO - 2, -1, -1):
            vals = jnp.where(idx == k, t[k], vals)
        out_v[pl.ds(i * _L, _L)] = vals
    pltpu.sync_copy(out_v, out_hbm.at[pl.ds(base, _BPW)])


@functools.lru_cache(maxsize=1)
def _lookup_call():
    return pl.kernel(
        _lookup_body,
        mesh=plsc.VectorSubcoreMesh(core_axis_name="c", subcore_axis_name="s"),
        out_type=jax.ShapeDtypeStruct((_B,), jnp.float32),
        scratch_types=[
            pltpu.VMEM((8, _L), jnp.float32),
            pltpu.VMEM((_BPW,), jnp.int32),
            pltpu.VMEM((_BPW,), jnp.int32),
            pltpu.VMEM((_BPW,), jnp.float32),
        ],
    )


def kernel(store, loc, store_emb, loc_emb, W1, b1, W2, b2):
    table = _table_call(store_emb, loc_emb, W1,
                        b1.reshape(1, -1), W2, b2.reshape(1, -1))
    out = _lookup_call()(table,
                         store.astype(jnp.int32), loc.astype(jnp.int32))
    return out.reshape(_B, 1)


# speedup vs baseline: 4.8782x; 1.0461x over previous
import functools

import jax
import jax.numpy as jnp
from jax import lax
from jax.experimental import pallas as pl
from jax.experimental.pallas import tpu as pltpu
from jax.experimental.pallas import tpu_sc as plsc

_B = 16384
_STORE_VOCAB = 2
_LOC_VOCAB = 3
_NCOMBO = _STORE_VOCAB * _LOC_VOCAB

_NC, _NS, _L = 2, 16, 16
_NW = _NC * _NS
_BPW = _B // _NW


def _table_body(se, le, w1, b1, w2, b2, out):
    xs = jnp.repeat(se[...], _LOC_VOCAB, axis=0)
    xl = jnp.tile(le[...], (_STORE_VOCAB, 1))
    x = jnp.concatenate([xs, xl], axis=1)
    h = jnp.maximum(
        jnp.dot(x, w1[...], preferred_element_type=jnp.float32) + b1[...], 0.0)
    o = jnp.maximum(
        jnp.dot(h, w2[...], preferred_element_type=jnp.float32) + b2[...], 0.0)
    o16 = jnp.broadcast_to(o, (_NCOMBO, _L))
    out[...] = jnp.concatenate(
        [o16, jnp.zeros((8 - _NCOMBO, _L), jnp.float32)], axis=0)


_table_call = pl.pallas_call(
    _table_body,
    out_shape=jax.ShapeDtypeStruct((8, _L), jnp.float32),
)


def _lookup_body(table_hbm, store_hbm, loc_hbm, out_hbm,
                 table_v, store_v, loc_v, out_v, sem):
    wid = lax.axis_index("s") * _NC + lax.axis_index("c")
    base = wid * _BPW
    cp_t = pltpu.async_copy(table_hbm, table_v, sem)
    cp_s = pltpu.async_copy(store_hbm.at[pl.ds(base, _BPW)], store_v, sem)
    cp_l = pltpu.async_copy(loc_hbm.at[pl.ds(base, _BPW)], loc_v, sem)
    cp_t.wait()
    cp_s.wait()
    cp_l.wait()
    t = [table_v[k] for k in range(_NCOMBO)]
    for i in range(_BPW // _L):
        s = store_v[pl.ds(i * _L, _L)]
        l = loc_v[pl.ds(i * _L, _L)]
        idx = s * _LOC_VOCAB + l
        vals = t[_NCOMBO - 1]
        for k in range(_NCOMBO - 2, -1, -1):
            vals = jnp.where(idx == k, t[k], vals)
        out_v[pl.ds(i * _L, _L)] = vals
    pltpu.sync_copy(out_v, out_hbm.at[pl.ds(base, _BPW)])


@functools.lru_cache(maxsize=1)
def _lookup_call():
    return pl.kernel(
        _lookup_body,
        mesh=plsc.VectorSubcoreMesh(core_axis_name="c", subcore_axis_name="s"),
        out_type=jax.ShapeDtypeStruct((_B,), jnp.float32),
        scratch_types=[
            pltpu.VMEM((8, _L), jnp.float32),
            pltpu.VMEM((_BPW,), jnp.int32),
            pltpu.VMEM((_BPW,), jnp.int32),
            pltpu.VMEM((_BPW,), jnp.float32),
            pltpu.SemaphoreType.DMA,
        ],
    )


def kernel(store, loc, store_emb, loc_emb, W1, b1, W2, b2):
    table = _table_call(store_emb, loc_emb, W1,
                        b1.reshape(1, -1), W2, b2.reshape(1, -1))
    out = _lookup_call()(table,
                         store.astype(jnp.int32), loc.astype(jnp.int32))
    return out.reshape(_B, 1)


# speedup vs baseline: 4.8946x; 1.0034x over previous
import functools

import jax
import jax.numpy as jnp
from jax import lax
from jax.experimental import pallas as pl
from jax.experimental.pallas import tpu as pltpu
from jax.experimental.pallas import tpu_sc as plsc

_B = 16384
_STORE_VOCAB = 2
_LOC_VOCAB = 3
_NCOMBO = _STORE_VOCAB * _LOC_VOCAB

_NC, _NS, _L = 2, 16, 16
_NW = _NC * _NS
_BPW = _B // _NW


def _table_body(se, le, w1, b1, w2, b2, out):
    xs = jnp.repeat(se[...], _LOC_VOCAB, axis=0)
    xl = jnp.tile(le[...], (_STORE_VOCAB, 1))
    x = jnp.concatenate([xs, xl], axis=1)
    h = jnp.maximum(
        jnp.dot(x, w1[...], preferred_element_type=jnp.float32) + b1[...], 0.0)
    o = jnp.maximum(
        jnp.dot(h, w2[...], preferred_element_type=jnp.float32) + b2[...], 0.0)
    o16 = jnp.broadcast_to(o, (_NCOMBO, _L))
    out[...] = jnp.concatenate(
        [o16, jnp.zeros((8 - _NCOMBO, _L), jnp.float32)], axis=0)


_table_call = pl.pallas_call(
    _table_body,
    out_shape=jax.ShapeDtypeStruct((8, _L), jnp.float32),
)


def _lookup_body(table_hbm, store_hbm, loc_hbm, out_hbm,
                 table_v, store_v, loc_v, out_v, sem):
    wid = lax.axis_index("s") * _NC + lax.axis_index("c")
    base = wid * _BPW
    cp_t = pltpu.async_copy(table_hbm, table_v, sem)
    cp_s = pltpu.async_copy(store_hbm.at[pl.ds(base, _BPW)], store_v, sem)
    cp_l = pltpu.async_copy(loc_hbm.at[pl.ds(base, _BPW)], loc_v, sem)
    cp_t.wait()
    cp_s.wait()
    cp_l.wait()
    t = [table_v[k] for k in range(_NCOMBO)]
    half = _BPW // 2

    def _groups(lo, hi):
        for i in range(lo, hi):
            s = store_v[pl.ds(i * _L, _L)]
            l = loc_v[pl.ds(i * _L, _L)]
            l0 = l == 0
            l1 = l == 1
            a = jnp.where(l0, t[0], jnp.where(l1, t[1], t[2]))
            b = jnp.where(l0, t[3], jnp.where(l1, t[4], t[5]))
            out_v[pl.ds(i * _L, _L)] = jnp.where(s == 0, a, b)

    _groups(0, half // _L)
    cp_o0 = pltpu.async_copy(out_v.at[pl.ds(0, half)],
                             out_hbm.at[pl.ds(base, half)], sem)
    _groups(half // _L, _BPW // _L)
    cp_o1 = pltpu.async_copy(out_v.at[pl.ds(half, half)],
                             out_hbm.at[pl.ds(base + half, half)], sem)
    cp_o0.wait()
    cp_o1.wait()


@functools.lru_cache(maxsize=1)
def _lookup_call():
    return pl.kernel(
        _lookup_body,
        mesh=plsc.VectorSubcoreMesh(core_axis_name="c", subcore_axis_name="s"),
        out_type=jax.ShapeDtypeStruct((_B,), jnp.float32),
        scratch_types=[
            pltpu.VMEM((8, _L), jnp.float32),
            pltpu.VMEM((_BPW,), jnp.int32),
            pltpu.VMEM((_BPW,), jnp.int32),
            pltpu.VMEM((_BPW,), jnp.float32),
            pltpu.SemaphoreType.DMA,
        ],
    )


def kernel(store, loc, store_emb, loc_emb, W1, b1, W2, b2):
    table = _table_call(store_emb, loc_emb, W1,
                        b1.reshape(1, -1), W2, b2.reshape(1, -1))
    out = _lookup_call()(table,
                         store.astype(jnp.int32), loc.astype(jnp.int32))
    return out.reshape(_B, 1)
